# Initial kernel scaffold; baseline (speedup 1.0000x reference)
import functools
import math

import jax
import jax.numpy as jnp
from jax import lax
from jax.experimental import pallas as pl

_SIZE = 32000
_PAD_IDX = 0
_SMOOTHING = 0.1
_CONFIDENCE = 1.0 - _SMOOTHING
_S = _SMOOTHING / (_SIZE - 2)

_BR = 64


def _ls_kernel(t_ref, x_ref, o_ref, *, c0):
    ri = pl.program_id(0)

    @pl.when(ri == 0)
    def _init():
        o_ref[0, 0] = 0.0

    x = x_ref[...]
    t = t_ref[...]
    mask = (t != _PAD_IDX)
    cols = lax.broadcasted_iota(jnp.int32, x.shape, 1)
    coef = jnp.float32(-_S) \
        + jnp.where(cols == _PAD_IDX, jnp.float32(_S), 0.0) \
        + jnp.where(cols == t, jnp.float32(_S - _CONFIDENCE), 0.0)
    coef = jnp.where(mask, coef, 0.0)
    part = jnp.sum(x * coef) + jnp.float32(c0) * jnp.sum(mask.astype(jnp.float32))
    o_ref[0, 0] += part


def kernel(x, target):
    n, v = x.shape
    c0 = _SMOOTHING * math.log(_S) + _CONFIDENCE * math.log(_CONFIDENCE)
    t2 = target.reshape(n, 1)
    grid = (n // _BR,)
    out = pl.pallas_call(
        functools.partial(_ls_kernel, c0=c0),
        grid=grid,
        in_specs=[
            pl.BlockSpec((_BR, 1), lambda i: (i, 0)),
            pl.BlockSpec((_BR, v), lambda i: (i, 0)),
        ],
        out_specs=pl.BlockSpec((1, 1), lambda i: (0, 0)),
        out_shape=jax.ShapeDtypeStruct((1, 1), jnp.float32),
    )(t2, x)
    return out.reshape(())


# speedup vs baseline: 7.1286x; 7.1286x over previous
import functools
import math

import jax
import jax.numpy as jnp
from jax import lax
from jax.experimental import pallas as pl

_SIZE = 32000
_PAD_IDX = 0
_SMOOTHING = 0.1
_CONFIDENCE = 1.0 - _SMOOTHING
_S = _SMOOTHING / (_SIZE - 2)

_BR = 64


def _ls_kernel(t_ref, x_ref, o_ref, *, c0):
    ri = pl.program_id(0)

    @pl.when(ri == 0)
    def _init():
        o_ref[...] = jnp.zeros_like(o_ref)

    x = x_ref[...]
    t = t_ref[...]
    mask = (t != _PAD_IDX)
    cols = lax.broadcasted_iota(jnp.int32, x.shape, 1)
    coef = jnp.float32(-_S) \
        + jnp.where(cols == _PAD_IDX, jnp.float32(_S), 0.0) \
        + jnp.where(cols == t, jnp.float32(_S - _CONFIDENCE), 0.0)
    coef = jnp.where(mask, coef, 0.0)
    part = jnp.sum(x * coef) + jnp.float32(c0) * jnp.sum(mask.astype(jnp.float32))
    o_ref[...] += part.reshape(1, 1)


def kernel(x, target):
    n, v = x.shape
    c0 = _SMOOTHING * math.log(_S) + _CONFIDENCE * math.log(_CONFIDENCE)
    t2 = target.reshape(n, 1)
    grid = (n // _BR,)
    out = pl.pallas_call(
        functools.partial(_ls_kernel, c0=c0),
        grid=grid,
        in_specs=[
            pl.BlockSpec((_BR, 1), lambda i: (i, 0)),
            pl.BlockSpec((_BR, v), lambda i: (i, 0)),
        ],
        out_specs=pl.BlockSpec((1, 1), lambda i: (0, 0)),
        out_shape=jax.ShapeDtypeStruct((1, 1), jnp.float32),
    )(t2, x)
    return out.reshape(())
